# baseline (speedup 1.0000x reference)
import jax
import jax.numpy as jnp
from jax.experimental import pallas as pl

LAM = 0.5


def _relu_pallas(x):
    def body(x_ref, o_ref):
        o_ref[...] = jnp.maximum(x_ref[...], 0.0)
    return pl.pallas_call(
        body, out_shape=jax.ShapeDtypeStruct(x.shape, x.dtype)
    )(x)


def _spmm(edge_index, vals, x):
    msg = vals[:, None] * jnp.take(x, edge_index[1], axis=0)
    return jnp.zeros((x.shape[0], x.shape[1]), dtype=x.dtype).at[edge_index[0]].add(msg)


def kernel(edge_index, adj_vals, train_sample, rna, atac, W_rna1, W_rna2, W_atac1, W_atac2, mlp_w1, mlp_b1, mlp_w2):
    h1 = _relu_pallas(_spmm(edge_index, adj_vals, rna @ W_rna1))
    h1 = _relu_pallas(_spmm(edge_index, adj_vals, h1 @ W_rna2))
    h2 = _relu_pallas(_spmm(edge_index, adj_vals, atac @ W_atac1))
    h2 = _relu_pallas(_spmm(edge_index, adj_vals, h2 @ W_atac2))
    h = (1.0 - LAM) * h1 + LAM * h2
    e1 = jnp.take(h, train_sample[:, 0], axis=0)
    e2 = jnp.take(h, train_sample[:, 1], axis=0)
    x = jnp.concatenate([e1, e2], axis=1)
    x = jax.nn.relu(x @ mlp_w1 + mlp_b1)
    pred = x @ mlp_w2
    return jax.nn.relu(pred)


# speedup vs baseline: 2.8620x; 2.8620x over previous
import functools

import jax
import jax.numpy as jnp
from jax import lax
from jax.experimental import pallas as pl
from jax.experimental.pallas import tpu as pltpu
from jax.experimental.pallas import tpu_sc as plsc

N = 10000
E = 320000
D = 128
H1 = 256
EMB = 128
MLP_H = 64
B = 16384
LAM = 0.5

NC = 2
NS = 16
FCH = 128
EPT = E // NS
G = 80
NP = 10240
RPT = NP // NS
ZR = 128

_mesh = plsc.VectorSubcoreMesh(core_axis_name="c", subcore_axis_name="s")


def _make_spmm(C):
    CPS = C // NC

    @functools.partial(
        pl.kernel,
        out_type=jax.ShapeDtypeStruct((C * NP, FCH), jnp.float32),
        mesh=_mesh,
        scratch_types=[
            pltpu.VMEM((G,), jnp.int32),
            pltpu.VMEM((G,), jnp.int32),
            pltpu.VMEM((G,), jnp.float32),
            pltpu.VMEM((G, FCH), jnp.float32),
            pltpu.VMEM((ZR, FCH), jnp.float32),
            pltpu.VMEM_SHARED((NP, FCH), jnp.float32),
            pltpu.SemaphoreType.DMA,
        ],
    )
    def spmm(x_hbm, src_hbm, dst_hbm, val_hbm, out_hbm,
             src_v, dst_v, val_v, rows_v, zbuf, acc, sem):
        cid = lax.axis_index("c")
        sid = lax.axis_index("s")
        zv = jnp.zeros((16,), jnp.float32)

        def zrow(i, carry):
            for k in range(FCH // 16):
                zbuf[i, pl.ds(k * 16, 16)] = zv
            return carry

        lax.fori_loop(0, ZR, zrow, 0)

        ebase = sid * EPT
        rbase = sid * RPT
        for j in range(CPS):
            c = cid * CPS + j
            goff = c * N
            row0 = c * NP
            for z in range(RPT // ZR):
                pltpu.sync_copy(zbuf, acc.at[pl.ds(rbase + z * ZR, ZR)])
            plsc.subcore_barrier()

            def chunk(g, carry):
                base = ebase + g * G
                pltpu.sync_copy(src_hbm.at[pl.ds(base, G)], src_v)
                pltpu.sync_copy(dst_hbm.at[pl.ds(base, G)], dst_v)
                pltpu.sync_copy(val_hbm.at[pl.ds(base, G)], val_v)
                off = jnp.full((16,), goff, jnp.int32)

                def addoff(i, cy):
                    sl = pl.ds(i * 16, 16)
                    src_v[sl] = src_v[sl] + off
                    return cy

                lax.fori_loop(0, G // 16, addoff, 0)
                pltpu.async_copy(x_hbm.at[src_v], rows_v, sem).wait()

                def edge16(gi, cy):
                    vv = val_v[pl.ds(gi * 16, 16)]
                    for l in range(16):
                        v = vv[l]
                        e = gi * 16 + l
                        for k in range(FCH // 16):
                            sl = pl.ds(k * 16, 16)
                            rows_v[e, sl] = rows_v[e, sl] * v
                    return cy

                lax.fori_loop(0, G // 16, edge16, 0)
                pltpu.sync_copy(rows_v, acc.at[dst_v], add=True)
                return carry

            lax.fori_loop(0, EPT // G, chunk, 0)
            plsc.subcore_barrier()
            for z in range(RPT // ZR):
                sl = pl.ds(rbase + z * ZR, ZR)
                pltpu.sync_copy(acc.at[sl], out_hbm.at[pl.ds(row0 + rbase + z * ZR, ZR)])
            if j + 1 < CPS:
                plsc.subcore_barrier()

    return spmm


_spmm4 = _make_spmm(4)
_spmm2 = _make_spmm(2)

PG = 2 * B // (NC * NS)
PGC = 256


@functools.partial(
    pl.kernel,
    out_type=jax.ShapeDtypeStruct((2 * B, EMB), jnp.float32),
    mesh=_mesh,
    scratch_types=[
        pltpu.VMEM((PGC,), jnp.int32),
        pltpu.VMEM((PGC, EMB), jnp.float32),
        pltpu.SemaphoreType.DMA,
    ],
)
def _pair_gather(h_hbm, idx_hbm, out_hbm, idx_v, rows_v, sem):
    wid = lax.axis_index("s") * NC + lax.axis_index("c")
    base = wid * PG

    def it(g, carry):
        b0 = base + g * PGC
        pltpu.sync_copy(idx_hbm.at[pl.ds(b0, PGC)], idx_v)
        pltpu.async_copy(h_hbm.at[idx_v], rows_v, sem).wait()
        pltpu.sync_copy(rows_v, out_hbm.at[pl.ds(b0, PGC)])
        return carry

    lax.fori_loop(0, PG // PGC, it, 0)


_NT = 5
_RB = N // _NT


def _mm_in_body(x_ref, w_ref, o_ref):
    o_ref[...] = jnp.dot(x_ref[0], w_ref[0],
                         preferred_element_type=jnp.float32)[None, None]


def _mm_in(x, w):
    return pl.pallas_call(
        _mm_in_body,
        grid=(2, _NT, 2),
        in_specs=[
            pl.BlockSpec((1, _RB, D), lambda b, i, h: (b, i, 0)),
            pl.BlockSpec((1, D, FCH), lambda b, i, h: (b, 0, h)),
        ],
        out_specs=pl.BlockSpec((1, 1, _RB, FCH), lambda b, i, h: (b, h, i, 0)),
        out_shape=jax.ShapeDtypeStruct((2, 2, N, FCH), jnp.float32),
    )(x, w)


def _mm_mid_body(h_ref, w_ref, o_ref):
    a = jnp.maximum(h_ref[...], 0.0)
    w = w_ref[0]
    o_ref[...] = (jnp.dot(a[0], w[:FCH], preferred_element_type=jnp.float32)
                  + jnp.dot(a[1], w[FCH:], preferred_element_type=jnp.float32))[None]


def _mm_mid(h1, w):
    return pl.pallas_call(
        _mm_mid_body,
        grid=(2, _NT),
        in_specs=[
            pl.BlockSpec((2, _RB, FCH), lambda b, i: (b, i, 0)),
            pl.BlockSpec((1, H1, EMB), lambda b, i: (b, 0, 0)),
        ],
        out_specs=pl.BlockSpec((1, _RB, EMB), lambda b, i: (b, i, 0)),
        out_shape=jax.ShapeDtypeStruct((2, N, EMB), jnp.float32),
    )(h1, w)


def _combine_body(h_ref, o_ref):
    a = h_ref[...]
    o_ref[...] = ((1.0 - LAM) * jnp.maximum(a[0], 0.0)
                  + LAM * jnp.maximum(a[1], 0.0))


def _combine(h2):
    return pl.pallas_call(
        _combine_body,
        grid=(_NT,),
        in_specs=[pl.BlockSpec((2, _RB, EMB), lambda i: (0, i, 0))],
        out_specs=pl.BlockSpec((_RB, EMB), lambda i: (i, 0)),
        out_shape=jax.ShapeDtypeStruct((N, EMB), jnp.float32),
    )(h2)


_BB = 2048


def _mlp_body(xp_ref, w1_ref, b1_ref, w2_ref, o_ref):
    e = xp_ref[...]
    hh = (jnp.dot(e[0], w1_ref[:EMB], preferred_element_type=jnp.float32)
          + jnp.dot(e[1], w1_ref[EMB:], preferred_element_type=jnp.float32)
          + b1_ref[...])
    hh = jnp.maximum(hh, 0.0)
    p = jnp.dot(hh, w2_ref[...], preferred_element_type=jnp.float32)
    o_ref[...] = jnp.maximum(p, 0.0)


def _mlp(xp, w1, b1, w2):
    return pl.pallas_call(
        _mlp_body,
        grid=(B // _BB,),
        in_specs=[
            pl.BlockSpec((2, _BB, EMB), lambda i: (0, i, 0)),
            pl.BlockSpec((2 * EMB, MLP_H), lambda i: (0, 0)),
            pl.BlockSpec((1, MLP_H), lambda i: (0, 0)),
            pl.BlockSpec((MLP_H, 1), lambda i: (0, 0)),
        ],
        out_specs=pl.BlockSpec((_BB, 1), lambda i: (i, 0)),
        out_shape=jax.ShapeDtypeStruct((B, 1), jnp.float32),
    )(xp, w1, b1, w2)


def kernel(edge_index, adj_vals, train_sample, rna, atac,
           W_rna1, W_rna2, W_atac1, W_atac2, mlp_w1, mlp_b1, mlp_w2):
    src = edge_index[1]
    dst = edge_index[0]
    x_in = jnp.stack([rna, atac])
    w1s = jnp.stack([W_rna1, W_atac1])
    w2s = jnp.stack([W_rna2, W_atac2])

    X1 = _mm_in(x_in, w1s)
    H1f = _spmm4(X1.reshape(4 * N, FCH), src, dst, adj_vals)
    X2 = _mm_mid(H1f.reshape(4, NP, FCH), w2s)
    H2f = _spmm2(X2.reshape(2 * N, FCH), src, dst, adj_vals)
    h = _combine(H2f.reshape(2, NP, EMB))
    idxp = train_sample.T.reshape(-1)
    xp = _pair_gather(h, idxp)
    return _mlp(xp.reshape(2, B, EMB), mlp_w1, mlp_b1.reshape(1, MLP_H), mlp_w2)


# speedup vs baseline: 5.9547x; 2.0806x over previous
import functools

import jax
import jax.numpy as jnp
from jax import lax
from jax.experimental import pallas as pl
from jax.experimental.pallas import tpu as pltpu
from jax.experimental.pallas import tpu_sc as plsc

N = 10000
E = 320000
D = 128
H1 = 256
EMB = 128
MLP_H = 64
B = 16384
LAM = 0.5

NC = 2
NS = 16
FCH = 128
EPT = E // NS
G = 80
NIT = EPT // G
NBLK = E // G
BW3 = 3 * G
NP = 10240
RPT = NP // NS
ZR = 64

_mesh = plsc.VectorSubcoreMesh(core_axis_name="c", subcore_axis_name="s")


def _make_spmm(C):
    CPS = C // NC

    @functools.partial(
        pl.kernel,
        out_type=jax.ShapeDtypeStruct((C * NP, FCH), jnp.float32),
        mesh=_mesh,
        scratch_types=[
            pltpu.VMEM((BW3,), jnp.float32),
            pltpu.VMEM((BW3,), jnp.float32),
            pltpu.VMEM((G,), jnp.int32),
            pltpu.VMEM((G,), jnp.int32),
            pltpu.VMEM((G,), jnp.int32),
            pltpu.VMEM((G,), jnp.int32),
            pltpu.VMEM((G, FCH), jnp.float32),
            pltpu.VMEM((G, FCH), jnp.float32),
            pltpu.VMEM((ZR, FCH), jnp.float32),
            pltpu.VMEM_SHARED((NP, FCH), jnp.float32),
            pltpu.SemaphoreType.DMA,
            pltpu.SemaphoreType.DMA,
            pltpu.SemaphoreType.DMA,
            pltpu.SemaphoreType.DMA,
        ],
    )
    def spmm(x_hbm, ei_hbm, out_hbm,
             idx0, idx1, srci0, srci1, dsti0, dsti1,
             rows0, rows1, zbuf, acc, si0, si1, sg0, sg1):
        cid = lax.axis_index("c")
        sid = lax.axis_index("s")
        zv = jnp.zeros((16,), jnp.float32)

        def zrow(i, carry):
            for k in range(FCH // 16):
                zbuf[i, pl.ds(k * 16, 16)] = zv
            return carry

        lax.fori_loop(0, ZR, zrow, 0)

        b0 = sid * NIT
        rbase = sid * RPT

        def stage_i(idx_ref, sem, blk):
            pltpu.async_copy(ei_hbm.at[pl.ds(blk * BW3, BW3)], idx_ref, sem)

        def stage_p(idx_ref, sem, srci_ref, rows_ref, gsem, blk, off):
            pltpu.make_async_copy(
                ei_hbm.at[pl.ds(blk * BW3, BW3)], idx_ref, sem).wait()

            def addoff(i, cy):
                sl16 = pl.ds(G + i * 16, 16)
                srci_ref[pl.ds(i * 16, 16)] = (
                    idx_ref[sl16].astype(jnp.int32) + off)
                return cy

            lax.fori_loop(0, G // 16, addoff, 0)
            pltpu.async_copy(x_hbm.at[srci_ref], rows_ref, gsem)

        def stage_c(idx_ref, srci_ref, dsti_ref, rows_ref, gsem):
            pltpu.make_async_copy(x_hbm.at[srci_ref], rows_ref, gsem).wait()

            def edge16(gi, cy):
                dsti_ref[pl.ds(gi * 16, 16)] = (
                    idx_ref[pl.ds(gi * 16, 16)].astype(jnp.int32))
                vv = idx_ref[pl.ds(2 * G + gi * 16, 16)]
                for l in range(16):
                    v = vv[l]
                    e = gi * 16 + l
                    for k in range(FCH // 16):
                        sl = pl.ds(k * 16, 16)
                        rows_ref[e, sl] = rows_ref[e, sl] * v
                return cy

            lax.fori_loop(0, G // 16, edge16, 0)
            pltpu.sync_copy(rows_ref, acc.at[dsti_ref], add=True)

        for j in range(CPS):
            c = cid * CPS + j
            goff = c * N
            row0 = c * NP
            off = jnp.full((16,), goff, jnp.int32)
            for z in range(RPT // ZR):
                pltpu.sync_copy(zbuf, acc.at[pl.ds(rbase + z * ZR, ZR)])
            plsc.subcore_barrier()

            stage_i(idx0, si0, b0)
            stage_i(idx1, si1, b0 + 1)
            stage_p(idx0, si0, srci0, rows0, sg0, b0, off)

            def body(o2, carry):
                g0 = 2 * o2
                stage_p(idx1, si1, srci1, rows1, sg1, b0 + g0 + 1, off)
                stage_c(idx0, srci0, dsti0, rows0, sg0)
                stage_i(idx0, si0, b0 + g0 + 2)
                stage_p(idx0, si0, srci0, rows0, sg0, b0 + g0 + 2, off)
                stage_c(idx1, srci1, dsti1, rows1, sg1)
                stage_i(idx1, si1, b0 + g0 + 3)
                return carry

            lax.fori_loop(0, NIT // 2 - 1, body, 0)
            stage_p(idx1, si1, srci1, rows1, sg1, b0 + NIT - 1, off)
            stage_c(idx0, srci0, dsti0, rows0, sg0)
            stage_c(idx1, srci1, dsti1, rows1, sg1)

            plsc.subcore_barrier()
            pltpu.sync_copy(acc.at[pl.ds(rbase, RPT)],
                            out_hbm.at[pl.ds(row0 + rbase, RPT)])
            if j + 1 < CPS:
                plsc.subcore_barrier()

    return spmm


_spmm4 = _make_spmm(4)
_spmm2 = _make_spmm(2)

PG = 2 * B // (NC * NS)
PGC = 128


@functools.partial(
    pl.kernel,
    out_type=jax.ShapeDtypeStruct((2 * B, EMB), jnp.float32),
    mesh=_mesh,
    scratch_types=[
        pltpu.VMEM((PGC,), jnp.int32),
        pltpu.VMEM((PGC, EMB), jnp.float32),
        pltpu.SemaphoreType.DMA,
    ],
)
def _pair_gather(h_hbm, idx_hbm, out_hbm, idx_v, rows_v, sem):
    wid = lax.axis_index("s") * NC + lax.axis_index("c")
    base = wid * PG

    def it(g, carry):
        b0 = base + g * PGC
        pltpu.sync_copy(idx_hbm.at[pl.ds(b0, PGC)], idx_v)
        pltpu.async_copy(h_hbm.at[idx_v], rows_v, sem).wait()
        pltpu.sync_copy(rows_v, out_hbm.at[pl.ds(b0, PGC)])
        return carry

    lax.fori_loop(0, PG // PGC, it, 0)


_NT = 5
_RB = N // _NT


def _mm_in_body(x_ref, w_ref, o_ref):
    o_ref[...] = jnp.dot(x_ref[0], w_ref[0],
                         preferred_element_type=jnp.float32)[None, None]


def _mm_in(x, w):
    return pl.pallas_call(
        _mm_in_body,
        grid=(2, _NT, 2),
        in_specs=[
            pl.BlockSpec((1, _RB, D), lambda b, i, h: (b, i, 0)),
            pl.BlockSpec((1, D, FCH), lambda b, i, h: (b, 0, h)),
        ],
        out_specs=pl.BlockSpec((1, 1, _RB, FCH), lambda b, i, h: (b, h, i, 0)),
        out_shape=jax.ShapeDtypeStruct((2, 2, N, FCH), jnp.float32),
    )(x, w)


def _mm_mid_body(h_ref, w_ref, o_ref):
    a = jnp.maximum(h_ref[...], 0.0)
    w = w_ref[0]
    o_ref[...] = (jnp.dot(a[0], w[:FCH], preferred_element_type=jnp.float32)
                  + jnp.dot(a[1], w[FCH:], preferred_element_type=jnp.float32))[None]


def _mm_mid(h1, w):
    return pl.pallas_call(
        _mm_mid_body,
        grid=(2, _NT),
        in_specs=[
            pl.BlockSpec((2, _RB, FCH), lambda b, i: (b, i, 0)),
            pl.BlockSpec((1, H1, EMB), lambda b, i: (b, 0, 0)),
        ],
        out_specs=pl.BlockSpec((1, _RB, EMB), lambda b, i: (b, i, 0)),
        out_shape=jax.ShapeDtypeStruct((2, N, EMB), jnp.float32),
    )(h1, w)


def _combine_body(h_ref, o_ref):
    a = h_ref[...]
    o_ref[...] = ((1.0 - LAM) * jnp.maximum(a[0], 0.0)
                  + LAM * jnp.maximum(a[1], 0.0))


def _combine(h2):
    return pl.pallas_call(
        _combine_body,
        grid=(_NT,),
        in_specs=[pl.BlockSpec((2, _RB, EMB), lambda i: (0, i, 0))],
        out_specs=pl.BlockSpec((_RB, EMB), lambda i: (i, 0)),
        out_shape=jax.ShapeDtypeStruct((N, EMB), jnp.float32),
    )(h2)


_BB = 2048


def _mlp_body(xp_ref, w1_ref, b1_ref, w2_ref, o_ref):
    e = xp_ref[...]
    hh = (jnp.dot(e[0], w1_ref[:EMB], preferred_element_type=jnp.float32)
          + jnp.dot(e[1], w1_ref[EMB:], preferred_element_type=jnp.float32)
          + b1_ref[...])
    hh = jnp.maximum(hh, 0.0)
    p = jnp.dot(hh, w2_ref[...], preferred_element_type=jnp.float32)
    o_ref[...] = jnp.maximum(p, 0.0)


def _mlp(xp, w1, b1, w2):
    return pl.pallas_call(
        _mlp_body,
        grid=(B // _BB,),
        in_specs=[
            pl.BlockSpec((2, _BB, EMB), lambda i: (0, i, 0)),
            pl.BlockSpec((2 * EMB, MLP_H), lambda i: (0, 0)),
            pl.BlockSpec((1, MLP_H), lambda i: (0, 0)),
            pl.BlockSpec((MLP_H, 1), lambda i: (0, 0)),
        ],
        out_specs=pl.BlockSpec((_BB, 1), lambda i: (i, 0)),
        out_shape=jax.ShapeDtypeStruct((B, 1), jnp.float32),
    )(xp, w1, b1, w2)


def kernel(edge_index, adj_vals, train_sample, rna, atac,
           W_rna1, W_rna2, W_atac1, W_atac2, mlp_w1, mlp_b1, mlp_w2):
    ei3 = jnp.stack([edge_index[0].astype(jnp.float32),
                     edge_index[1].astype(jnp.float32), adj_vals])
    eiB = ei3.reshape(3, NBLK, G).transpose(1, 0, 2).reshape(-1)
    x_in = jnp.stack([rna, atac])
    w1s = jnp.stack([W_rna1, W_atac1])
    w2s = jnp.stack([W_rna2, W_atac2])

    X1 = _mm_in(x_in, w1s)
    H1f = _spmm4(X1.reshape(4 * N, FCH), eiB)
    X2 = _mm_mid(H1f.reshape(4, NP, FCH), w2s)
    H2f = _spmm2(X2.reshape(2 * N, FCH), eiB)
    h = _combine(H2f.reshape(2, NP, EMB))
    idxp = train_sample.T.reshape(-1)
    xp = _pair_gather(h, idxp)
    return _mlp(xp.reshape(2, B, EMB), mlp_w1, mlp_b1.reshape(1, MLP_H), mlp_w2)


# speedup vs baseline: 7.2140x; 1.2115x over previous
import functools

import jax
import jax.numpy as jnp
from jax import lax
from jax.experimental import pallas as pl
from jax.experimental.pallas import tpu as pltpu
from jax.experimental.pallas import tpu_sc as plsc

N = 10000
E = 320000
D = 128
H1 = 256
EMB = 128
MLP_H = 64
B = 16384
LAM = 0.5

NC = 2
NS = 16
FCH = 128
EPT = E // NS
G = 80
NIT = EPT // G
NBLK = E // G
BW3 = 3 * G
NP = 10240
RPT = NP // NS

_mesh = plsc.VectorSubcoreMesh(core_axis_name="c", subcore_axis_name="s")


def _make_spmm(C):
    CPS = C // NC

    @functools.partial(
        pl.kernel,
        out_type=jax.ShapeDtypeStruct((C * NP, FCH), jnp.float32),
        mesh=_mesh,
        scratch_types=[
            pltpu.VMEM((BW3,), jnp.float32),
            pltpu.VMEM((BW3,), jnp.float32),
            pltpu.VMEM((G,), jnp.int32),
            pltpu.VMEM((G,), jnp.int32),
            pltpu.VMEM((G,), jnp.int32),
            pltpu.VMEM((G,), jnp.int32),
            pltpu.VMEM((G, FCH), jnp.float32),
            pltpu.VMEM((G, FCH), jnp.float32),
            pltpu.VMEM((G, FCH), jnp.float32),
            pltpu.VMEM((G, FCH), jnp.float32),
            pltpu.VMEM_SHARED((NP, FCH), jnp.float32),
            pltpu.SemaphoreType.DMA,
            pltpu.SemaphoreType.DMA,
            pltpu.SemaphoreType.DMA,
            pltpu.SemaphoreType.DMA,
            pltpu.SemaphoreType.DMA,
            pltpu.SemaphoreType.DMA,
        ],
    )
    def spmm(x_hbm, ei_hbm, z_hbm, out_hbm,
             idx0, idx1, srci0, srci1, dsti0, dsti1,
             rows0, rows1, sbuf0, sbuf1, acc,
             si0, si1, sg0, sg1, ss0, ss1):
        cid = lax.axis_index("c")
        sid = lax.axis_index("s")

        b0 = sid * NIT
        rbase = sid * RPT

        def stage_i(idx_ref, sem, blk):
            pltpu.async_copy(ei_hbm.at[pl.ds(blk * BW3, BW3)], idx_ref, sem)

        def stage_p(idx_ref, sem, srci_ref, rows_ref, gsem, blk, off):
            pltpu.make_async_copy(
                ei_hbm.at[pl.ds(blk * BW3, BW3)], idx_ref, sem).wait()

            def addoff(i, cy):
                sl16 = pl.ds(G + i * 16, 16)
                srci_ref[pl.ds(i * 16, 16)] = (
                    idx_ref[sl16].astype(jnp.int32) + off)
                return cy

            lax.fori_loop(0, G // 16, addoff, 0)
            pltpu.async_copy(x_hbm.at[srci_ref], rows_ref, gsem)

        def scat_wait(sbuf_ref, dsti_ref, ssem):
            pltpu.make_async_copy(sbuf_ref, acc.at[dsti_ref], ssem).wait()

        def stage_c(idx_ref, srci_ref, dsti_ref, rows_ref, sbuf_ref,
                    gsem, ssem, guard_cond):
            pltpu.make_async_copy(x_hbm.at[srci_ref], rows_ref, gsem).wait()
            if guard_cond is None:
                scat_wait(sbuf_ref, dsti_ref, ssem)
            else:
                @pl.when(guard_cond)
                def _():
                    scat_wait(sbuf_ref, dsti_ref, ssem)

            def edge16(gi, cy):
                dsti_ref[pl.ds(gi * 16, 16)] = (
                    idx_ref[pl.ds(gi * 16, 16)].astype(jnp.int32))
                vv = idx_ref[pl.ds(2 * G + gi * 16, 16)]
                for l in range(16):
                    v = vv[l]
                    e = gi * 16 + l
                    for k in range(FCH // 16):
                        sl = pl.ds(k * 16, 16)
                        sbuf_ref[e, sl] = rows_ref[e, sl] * v
                return cy

            lax.fori_loop(0, G // 16, edge16, 0)
            pltpu.async_copy(sbuf_ref, acc.at[dsti_ref], ssem, add=True)

        for j in range(CPS):
            c = cid * CPS + j
            goff = c * N
            row0 = c * NP
            off = jnp.full((16,), goff, jnp.int32)
            pltpu.sync_copy(z_hbm, acc.at[pl.ds(rbase, RPT)])
            plsc.subcore_barrier()

            stage_i(idx0, si0, b0)
            stage_i(idx1, si1, b0 + 1)
            stage_p(idx0, si0, srci0, rows0, sg0, b0, off)

            def body(o2, carry):
                g0 = 2 * o2
                guard = o2 > 0
                stage_p(idx1, si1, srci1, rows1, sg1, b0 + g0 + 1, off)
                stage_c(idx0, srci0, dsti0, rows0, sbuf0, sg0, ss0, guard)
                stage_i(idx0, si0, b0 + g0 + 2)
                stage_p(idx0, si0, srci0, rows0, sg0, b0 + g0 + 2, off)
                stage_c(idx1, srci1, dsti1, rows1, sbuf1, sg1, ss1, guard)
                stage_i(idx1, si1, b0 + g0 + 3)
                return carry

            lax.fori_loop(0, NIT // 2 - 1, body, 0)
            stage_p(idx1, si1, srci1, rows1, sg1, b0 + NIT - 1, off)
            stage_c(idx0, srci0, dsti0, rows0, sbuf0, sg0, ss0, None)
            stage_c(idx1, srci1, dsti1, rows1, sbuf1, sg1, ss1, None)
            scat_wait(sbuf0, dsti0, ss0)
            scat_wait(sbuf1, dsti1, ss1)

            plsc.subcore_barrier()
            pltpu.sync_copy(acc.at[pl.ds(rbase, RPT)],
                            out_hbm.at[pl.ds(row0 + rbase, RPT)])
            if j + 1 < CPS:
                plsc.subcore_barrier()

    return spmm


_spmm4 = _make_spmm(4)
_spmm2 = _make_spmm(2)

PG = 2 * B // (NC * NS)
PGC = 128


@functools.partial(
    pl.kernel,
    out_type=jax.ShapeDtypeStruct((2 * B, EMB), jnp.float32),
    mesh=_mesh,
    scratch_types=[
        pltpu.VMEM((PGC,), jnp.int32),
        pltpu.VMEM((PGC, EMB), jnp.float32),
        pltpu.SemaphoreType.DMA,
    ],
)
def _pair_gather(h_hbm, idx_hbm, out_hbm, idx_v, rows_v, sem):
    wid = lax.axis_index("s") * NC + lax.axis_index("c")
    base = wid * PG

    def it(g, carry):
        b0 = base + g * PGC
        pltpu.sync_copy(idx_hbm.at[pl.ds(b0, PGC)], idx_v)
        pltpu.async_copy(h_hbm.at[idx_v], rows_v, sem).wait()
        pltpu.sync_copy(rows_v, out_hbm.at[pl.ds(b0, PGC)])
        return carry

    lax.fori_loop(0, PG // PGC, it, 0)


_NT = 5
_RB = N // _NT


def _mm_in_body(x_ref, w_ref, o_ref):
    o_ref[...] = jnp.dot(x_ref[0], w_ref[0],
                         preferred_element_type=jnp.float32)[None, None]


def _mm_in(x, w):
    return pl.pallas_call(
        _mm_in_body,
        grid=(2, _NT, 2),
        in_specs=[
            pl.BlockSpec((1, _RB, D), lambda b, i, h: (b, i, 0)),
            pl.BlockSpec((1, D, FCH), lambda b, i, h: (b, 0, h)),
        ],
        out_specs=pl.BlockSpec((1, 1, _RB, FCH), lambda b, i, h: (b, h, i, 0)),
        out_shape=jax.ShapeDtypeStruct((2, 2, N, FCH), jnp.float32),
    )(x, w)


def _mm_mid_body(h_ref, w_ref, o_ref):
    a = jnp.maximum(h_ref[...], 0.0)
    w = w_ref[0]
    o_ref[...] = (jnp.dot(a[0], w[:FCH], preferred_element_type=jnp.float32)
                  + jnp.dot(a[1], w[FCH:], preferred_element_type=jnp.float32))[None]


def _mm_mid(h1, w):
    return pl.pallas_call(
        _mm_mid_body,
        grid=(2, _NT),
        in_specs=[
            pl.BlockSpec((2, _RB, FCH), lambda b, i: (b, i, 0)),
            pl.BlockSpec((1, H1, EMB), lambda b, i: (b, 0, 0)),
        ],
        out_specs=pl.BlockSpec((1, _RB, EMB), lambda b, i: (b, i, 0)),
        out_shape=jax.ShapeDtypeStruct((2, N, EMB), jnp.float32),
    )(h1, w)


def _combine_body(h_ref, o_ref):
    a = h_ref[...]
    o_ref[...] = ((1.0 - LAM) * jnp.maximum(a[0], 0.0)
                  + LAM * jnp.maximum(a[1], 0.0))


def _combine(h2):
    return pl.pallas_call(
        _combine_body,
        grid=(_NT,),
        in_specs=[pl.BlockSpec((2, _RB, EMB), lambda i: (0, i, 0))],
        out_specs=pl.BlockSpec((_RB, EMB), lambda i: (i, 0)),
        out_shape=jax.ShapeDtypeStruct((N, EMB), jnp.float32),
    )(h2)


_BB = 2048


def _mlp_body(xp_ref, w1_ref, b1_ref, w2_ref, o_ref):
    e = xp_ref[...]
    hh = (jnp.dot(e[0], w1_ref[:EMB], preferred_element_type=jnp.float32)
          + jnp.dot(e[1], w1_ref[EMB:], preferred_element_type=jnp.float32)
          + b1_ref[...])
    hh = jnp.maximum(hh, 0.0)
    p = jnp.dot(hh, w2_ref[...], preferred_element_type=jnp.float32)
    o_ref[...] = jnp.maximum(p, 0.0)


def _mlp(xp, w1, b1, w2):
    return pl.pallas_call(
        _mlp_body,
        grid=(B // _BB,),
        in_specs=[
            pl.BlockSpec((2, _BB, EMB), lambda i: (0, i, 0)),
            pl.BlockSpec((2 * EMB, MLP_H), lambda i: (0, 0)),
            pl.BlockSpec((1, MLP_H), lambda i: (0, 0)),
            pl.BlockSpec((MLP_H, 1), lambda i: (0, 0)),
        ],
        out_specs=pl.BlockSpec((_BB, 1), lambda i: (i, 0)),
        out_shape=jax.ShapeDtypeStruct((B, 1), jnp.float32),
    )(xp, w1, b1, w2)


def kernel(edge_index, adj_vals, train_sample, rna, atac,
           W_rna1, W_rna2, W_atac1, W_atac2, mlp_w1, mlp_b1, mlp_w2):
    ei3 = jnp.stack([edge_index[0].astype(jnp.float32),
                     edge_index[1].astype(jnp.float32), adj_vals])
    eiB = ei3.reshape(3, NBLK, G).transpose(1, 0, 2).reshape(-1)
    x_in = jnp.stack([rna, atac])
    w1s = jnp.stack([W_rna1, W_atac1])
    w2s = jnp.stack([W_rna2, W_atac2])

    zrows = jnp.zeros((RPT, FCH), jnp.float32)

    X1 = _mm_in(x_in, w1s)
    H1f = _spmm4(X1.reshape(4 * N, FCH), eiB, zrows)
    X2 = _mm_mid(H1f.reshape(4, NP, FCH), w2s)
    H2f = _spmm2(X2.reshape(2 * N, FCH), eiB, zrows)
    h = _combine(H2f.reshape(2, NP, EMB))
    idxp = train_sample.T.reshape(-1)
    xp = _pair_gather(h, idxp)
    return _mlp(xp.reshape(2, B, EMB), mlp_w1, mlp_b1.reshape(1, MLP_H), mlp_w2)


# speedup vs baseline: 7.2193x; 1.0007x over previous
import functools

import jax
import jax.numpy as jnp
from jax import lax
from jax.experimental import pallas as pl
from jax.experimental.pallas import tpu as pltpu
from jax.experimental.pallas import tpu_sc as plsc

N = 10000
E = 320000
D = 128
H1 = 256
EMB = 128
MLP_H = 64
B = 16384
LAM = 0.5

NC = 2
NS = 16
FCH = 128
EPT = E // NS
G = 80
NIT = EPT // G
NBLK = E // G
BW3 = 3 * G
NP = 10240
RPT = NP // NS

_mesh = plsc.VectorSubcoreMesh(core_axis_name="c", subcore_axis_name="s")


def _make_spmm(C):
    CPS = C // NC

    @functools.partial(
        pl.kernel,
        out_type=jax.ShapeDtypeStruct((C * NP, FCH), jnp.float32),
        mesh=_mesh,
        scratch_types=[
            pltpu.VMEM((BW3,), jnp.float32),
            pltpu.VMEM((BW3,), jnp.float32),
            pltpu.VMEM((G,), jnp.int32),
            pltpu.VMEM((G,), jnp.int32),
            pltpu.VMEM((G,), jnp.int32),
            pltpu.VMEM((G,), jnp.int32),
            pltpu.VMEM((G, FCH), jnp.float32),
            pltpu.VMEM((G, FCH), jnp.float32),
            pltpu.VMEM((G, FCH), jnp.float32),
            pltpu.VMEM((G, FCH), jnp.float32),
            pltpu.VMEM_SHARED((NP, FCH), jnp.float32),
            pltpu.SemaphoreType.DMA,
            pltpu.SemaphoreType.DMA,
            pltpu.SemaphoreType.DMA,
            pltpu.SemaphoreType.DMA,
            pltpu.SemaphoreType.DMA,
            pltpu.SemaphoreType.DMA,
        ],
    )
    def spmm(x_hbm, ei_hbm, z_hbm, out_hbm,
             idx0, idx1, srci0, srci1, dsti0, dsti1,
             rows0, rows1, sbuf0, sbuf1, acc,
             si0, si1, sg0, sg1, ss0, ss1):
        cid = lax.axis_index("c")
        sid = lax.axis_index("s")

        b0 = sid * NIT
        rbase = sid * RPT

        def stage_i(idx_ref, sem, blk):
            pltpu.async_copy(ei_hbm.at[pl.ds(blk * BW3, BW3)], idx_ref, sem)

        def stage_p(idx_ref, sem, srci_ref, rows_ref, gsem, blk, off):
            pltpu.make_async_copy(
                ei_hbm.at[pl.ds(blk * BW3, BW3)], idx_ref, sem).wait()

            @plsc.parallel_loop(0, G // 16)
            def addoff(i):
                sl16 = pl.ds(G + i * 16, 16)
                srci_ref[pl.ds(i * 16, 16)] = (
                    idx_ref[sl16].astype(jnp.int32) + off)
            pltpu.async_copy(x_hbm.at[srci_ref], rows_ref, gsem)

        def scat_wait(sbuf_ref, dsti_ref, ssem):
            pltpu.make_async_copy(sbuf_ref, acc.at[dsti_ref], ssem).wait()

        def stage_c(idx_ref, srci_ref, dsti_ref, rows_ref, sbuf_ref,
                    gsem, ssem, guard_cond):
            pltpu.make_async_copy(x_hbm.at[srci_ref], rows_ref, gsem).wait()
            if guard_cond is None:
                scat_wait(sbuf_ref, dsti_ref, ssem)
            else:
                @pl.when(guard_cond)
                def _():
                    scat_wait(sbuf_ref, dsti_ref, ssem)

            @plsc.parallel_loop(0, G // 16)
            def edge16(gi):
                dsti_ref[pl.ds(gi * 16, 16)] = (
                    idx_ref[pl.ds(gi * 16, 16)].astype(jnp.int32))
                vv = idx_ref[pl.ds(2 * G + gi * 16, 16)]
                for l in range(16):
                    v = vv[l]
                    e = gi * 16 + l
                    for k in range(FCH // 16):
                        sl = pl.ds(k * 16, 16)
                        sbuf_ref[e, sl] = rows_ref[e, sl] * v
            pltpu.async_copy(sbuf_ref, acc.at[dsti_ref], ssem, add=True)

        for j in range(CPS):
            c = cid * CPS + j
            goff = c * N
            row0 = c * NP
            off = jnp.full((16,), goff, jnp.int32)
            pltpu.sync_copy(z_hbm, acc.at[pl.ds(rbase, RPT)])
            plsc.subcore_barrier()

            stage_i(idx0, si0, b0)
            stage_i(idx1, si1, b0 + 1)
            stage_p(idx0, si0, srci0, rows0, sg0, b0, off)

            def body(o2, carry):
                g0 = 2 * o2
                guard = o2 > 0
                stage_p(idx1, si1, srci1, rows1, sg1, b0 + g0 + 1, off)
                stage_c(idx0, srci0, dsti0, rows0, sbuf0, sg0, ss0, guard)
                stage_i(idx0, si0, b0 + g0 + 2)
                stage_p(idx0, si0, srci0, rows0, sg0, b0 + g0 + 2, off)
                stage_c(idx1, srci1, dsti1, rows1, sbuf1, sg1, ss1, guard)
                stage_i(idx1, si1, b0 + g0 + 3)
                return carry

            lax.fori_loop(0, NIT // 2 - 1, body, 0)
            stage_p(idx1, si1, srci1, rows1, sg1, b0 + NIT - 1, off)
            stage_c(idx0, srci0, dsti0, rows0, sbuf0, sg0, ss0, None)
            stage_c(idx1, srci1, dsti1, rows1, sbuf1, sg1, ss1, None)
            scat_wait(sbuf0, dsti0, ss0)
            scat_wait(sbuf1, dsti1, ss1)

            plsc.subcore_barrier()
            pltpu.sync_copy(acc.at[pl.ds(rbase, RPT)],
                            out_hbm.at[pl.ds(row0 + rbase, RPT)])
            if j + 1 < CPS:
                plsc.subcore_barrier()

    return spmm


_spmm4 = _make_spmm(4)
_spmm2 = _make_spmm(2)

PG = 2 * B // (NC * NS)
PGC = 128


@functools.partial(
    pl.kernel,
    out_type=jax.ShapeDtypeStruct((2 * B, EMB), jnp.float32),
    mesh=_mesh,
    scratch_types=[
        pltpu.VMEM((PGC,), jnp.int32),
        pltpu.VMEM((PGC, EMB), jnp.float32),
        pltpu.SemaphoreType.DMA,
    ],
)
def _pair_gather(h_hbm, idx_hbm, out_hbm, idx_v, rows_v, sem):
    wid = lax.axis_index("s") * NC + lax.axis_index("c")
    base = wid * PG

    def it(g, carry):
        b0 = base + g * PGC
        pltpu.sync_copy(idx_hbm.at[pl.ds(b0, PGC)], idx_v)
        pltpu.async_copy(h_hbm.at[idx_v], rows_v, sem).wait()
        pltpu.sync_copy(rows_v, out_hbm.at[pl.ds(b0, PGC)])
        return carry

    lax.fori_loop(0, PG // PGC, it, 0)


_NT = 5
_RB = N // _NT


def _mm_in_body(x_ref, w_ref, o_ref):
    o_ref[...] = jnp.dot(x_ref[0], w_ref[0],
                         preferred_element_type=jnp.float32)[None, None]


def _mm_in(x, w):
    return pl.pallas_call(
        _mm_in_body,
        grid=(2, _NT, 2),
        in_specs=[
            pl.BlockSpec((1, _RB, D), lambda b, i, h: (b, i, 0)),
            pl.BlockSpec((1, D, FCH), lambda b, i, h: (b, 0, h)),
        ],
        out_specs=pl.BlockSpec((1, 1, _RB, FCH), lambda b, i, h: (b, h, i, 0)),
        out_shape=jax.ShapeDtypeStruct((2, 2, N, FCH), jnp.float32),
    )(x, w)


def _mm_mid_body(h_ref, w_ref, o_ref):
    a = jnp.maximum(h_ref[...], 0.0)
    w = w_ref[0]
    o_ref[...] = (jnp.dot(a[0], w[:FCH], preferred_element_type=jnp.float32)
                  + jnp.dot(a[1], w[FCH:], preferred_element_type=jnp.float32))[None]


def _mm_mid(h1, w):
    return pl.pallas_call(
        _mm_mid_body,
        grid=(2, _NT),
        in_specs=[
            pl.BlockSpec((2, _RB, FCH), lambda b, i: (b, i, 0)),
            pl.BlockSpec((1, H1, EMB), lambda b, i: (b, 0, 0)),
        ],
        out_specs=pl.BlockSpec((1, _RB, EMB), lambda b, i: (b, i, 0)),
        out_shape=jax.ShapeDtypeStruct((2, N, EMB), jnp.float32),
    )(h1, w)


def _combine_body(h_ref, o_ref):
    a = h_ref[...]
    o_ref[...] = ((1.0 - LAM) * jnp.maximum(a[0], 0.0)
                  + LAM * jnp.maximum(a[1], 0.0))


def _combine(h2):
    return pl.pallas_call(
        _combine_body,
        grid=(_NT,),
        in_specs=[pl.BlockSpec((2, _RB, EMB), lambda i: (0, i, 0))],
        out_specs=pl.BlockSpec((_RB, EMB), lambda i: (i, 0)),
        out_shape=jax.ShapeDtypeStruct((N, EMB), jnp.float32),
    )(h2)


_BB = 2048


def _mlp_body(xp_ref, w1_ref, b1_ref, w2_ref, o_ref):
    e = xp_ref[...]
    hh = (jnp.dot(e[0], w1_ref[:EMB], preferred_element_type=jnp.float32)
          + jnp.dot(e[1], w1_ref[EMB:], preferred_element_type=jnp.float32)
          + b1_ref[...])
    hh = jnp.maximum(hh, 0.0)
    p = jnp.dot(hh, w2_ref[...], preferred_element_type=jnp.float32)
    o_ref[...] = jnp.maximum(p, 0.0)


def _mlp(xp, w1, b1, w2):
    return pl.pallas_call(
        _mlp_body,
        grid=(B // _BB,),
        in_specs=[
            pl.BlockSpec((2, _BB, EMB), lambda i: (0, i, 0)),
            pl.BlockSpec((2 * EMB, MLP_H), lambda i: (0, 0)),
            pl.BlockSpec((1, MLP_H), lambda i: (0, 0)),
            pl.BlockSpec((MLP_H, 1), lambda i: (0, 0)),
        ],
        out_specs=pl.BlockSpec((_BB, 1), lambda i: (i, 0)),
        out_shape=jax.ShapeDtypeStruct((B, 1), jnp.float32),
    )(xp, w1, b1, w2)


def kernel(edge_index, adj_vals, train_sample, rna, atac,
           W_rna1, W_rna2, W_atac1, W_atac2, mlp_w1, mlp_b1, mlp_w2):
    ei3 = jnp.stack([edge_index[0].astype(jnp.float32),
                     edge_index[1].astype(jnp.float32), adj_vals])
    eiB = ei3.reshape(3, NBLK, G).transpose(1, 0, 2).reshape(-1)
    x_in = jnp.stack([rna, atac])
    w1s = jnp.stack([W_rna1, W_atac1])
    w2s = jnp.stack([W_rna2, W_atac2])

    zrows = jnp.zeros((RPT, FCH), jnp.float32)

    X1 = _mm_in(x_in, w1s)
    H1f = _spmm4(X1.reshape(4 * N, FCH), eiB, zrows)
    X2 = _mm_mid(H1f.reshape(4, NP, FCH), w2s)
    H2f = _spmm2(X2.reshape(2 * N, FCH), eiB, zrows)
    h = _combine(H2f.reshape(2, NP, EMB))
    idxp = train_sample.T.reshape(-1)
    xp = _pair_gather(h, idxp)
    return _mlp(xp.reshape(2, B, EMB), mlp_w1, mlp_b1.reshape(1, MLP_H), mlp_w2)
